# baseline (speedup 1.0000x reference)
import functools

import jax
import jax.numpy as jnp
from jax import lax
from jax.experimental import pallas as pl
from jax.experimental.pallas import tpu as pltpu
from jax.experimental.pallas import tpu_sc as plsc

N_NODES = 10000
N_EDGES = 320000
N_GRAPHS = 256
D_NODE = 128
D_EDGE = 16
D_GLOBAL = 177
BN_EPS = 1e-5

NC = 2
NS = 16
NW = NC * NS
EPW = N_EDGES // NW
C = 80
NCH = EPW // C
ZR = 624
ZTAIL = N_NODES - NS * ZR

EB = 4000
BN = 1000


def _make_sc_agg(D, packed):
    mesh = plsc.VectorSubcoreMesh(core_axis_name="c", subcore_axis_name="s",
                                  num_cores=NC, num_subcores=NS)

    NB = 4 if D == 128 else 5
    NG = NCH // NB
    TAIL = NCH - NG * NB
    scratch_types = [
            pltpu.VMEM((NB, C), jnp.int32),
            pltpu.VMEM((NB, C), jnp.int32),
            pltpu.VMEM((NB, C // 2, 128) if packed else (NB, C, D),
                       jnp.float32),
            pltpu.VMEM_SHARED((N_NODES, D), jnp.float32),
            pltpu.SemaphoreType.DMA,
            pltpu.SemaphoreType.DMA,
            pltpu.SemaphoreType.DMA,
            pltpu.SemaphoreType.DMA,
        ]
    if packed:
        scratch_types.insert(3, pltpu.VMEM((NB, C, D), jnp.float32))

    def _body_common(x_hbm, src_hbm, dst_hbm, e_hbm, zeros_hbm, out_hbm,
                     srcb, dstb, ebuf, xbuf, acc, esem, gsem, ssem, isem):
        c = lax.axis_index("c")
        s = lax.axis_index("s")
        w = c * NS + s

        pltpu.sync_copy(zeros_hbm.at[pl.ds(s * ZR, ZR)],
                        acc.at[pl.ds(s * ZR, ZR)])

        @pl.when(s == NS - 1)
        def _():
            pltpu.sync_copy(zeros_hbm.at[pl.ds(NS * ZR, ZTAIL)],
                            acc.at[pl.ds(NS * ZR, ZTAIL)])
        plsc.subcore_barrier()

        def run_chunks(j0, n):
            idescs = [
                pltpu.async_copy(src_hbm.at[pl.ds(w * NCH + j0, n)],
                                 srcb.at[pl.ds(0, n)], isem),
                pltpu.async_copy(dst_hbm.at[pl.ds(w * NCH + j0, n)],
                                 dstb.at[pl.ds(0, n)], isem),
            ]
            edescs = []
            for b in range(n):
                base = w * EPW + (j0 + b) * C
                if packed:
                    edescs.append(pltpu.async_copy(
                        e_hbm.at[pl.ds(base // 2, C // 2)], ebuf.at[b], esem))
                else:
                    edescs.append(pltpu.async_copy(
                        e_hbm.at[pl.ds(base, C)], ebuf.at[b], esem))
            idescs[0].wait()
            idescs[1].wait()
            gdescs = []
            for b in range(n):
                if packed:
                    gdescs.append(pltpu.async_copy(
                        x_hbm.at[srcb.at[b]], xbuf.at[b], gsem))
                else:
                    edescs[b].wait()
                    gdescs.append(pltpu.async_copy(
                        x_hbm.at[srcb.at[b]], ebuf.at[b], gsem, add=True))
            sdescs = []
            for b in range(n):
                gdescs[b].wait()
                if packed:
                    edescs[b].wait()

                    def rows2(rr, carry2, b=b):
                        for half in range(2):
                            for k in range(D // 16):
                                xs = (b, 2 * rr + half, pl.ds(k * 16, 16))
                                es = (b, rr, pl.ds(half * 64 + k * 16, 16))
                                xbuf[xs] = jnp.maximum(xbuf[xs] + ebuf[es],
                                                       0.0)
                        return carry2
                    lax.fori_loop(0, C // 2, rows2, 0)
                    src_buf = xbuf
                else:
                    rpi = 256 // D

                    def rows(r, carry2, b=b):
                        for u in range(rpi):
                            for k in range(D // 16):
                                sl = (b, r * rpi + u, pl.ds(k * 16, 16))
                                ebuf[sl] = jnp.maximum(ebuf[sl], 0.0)
                        return carry2
                    lax.fori_loop(0, C // rpi, rows, 0)
                    src_buf = ebuf
                sdescs.append(pltpu.async_copy(
                    src_buf.at[b], acc.at[dstb.at[b]], ssem, add=True))
            for b in range(n):
                sdescs[b].wait()

        def group(g, carry):
            run_chunks(g * NB, NB)
            return carry
        lax.fori_loop(0, NG, group, 0)
        if TAIL:
            run_chunks(NG * NB, TAIL)

        plsc.subcore_barrier()
        pltpu.sync_copy(acc.at[pl.ds(s * ZR, ZR)],
                        out_hbm.at[c, pl.ds(s * ZR, ZR)])

        @pl.when(s == NS - 1)
        def _():
            pltpu.sync_copy(acc.at[pl.ds(NS * ZR, ZTAIL)],
                            out_hbm.at[c, pl.ds(NS * ZR, ZTAIL)])

    deco = functools.partial(
        pl.kernel,
        out_type=jax.ShapeDtypeStruct((NC, N_NODES, D), jnp.float32),
        mesh=mesh,
        scratch_types=scratch_types,
        compiler_params=pltpu.CompilerParams(use_tc_tiling_on_sc=False),
    )
    if packed:
        @deco
        def sc_agg(x_hbm, src_hbm, dst_hbm, e_hbm, zeros_hbm, out_hbm,
                   srcb, dstb, ebuf, xbuf, acc, esem, gsem, ssem, isem):
            _body_common(x_hbm, src_hbm, dst_hbm, e_hbm, zeros_hbm, out_hbm,
                         srcb, dstb, ebuf, xbuf, acc, esem, gsem, ssem, isem)
    else:
        @deco
        def sc_agg(x_hbm, src_hbm, dst_hbm, e_hbm, zeros_hbm, out_hbm,
                   srcb, dstb, ebuf, acc, esem, gsem, ssem, isem):
            _body_common(x_hbm, src_hbm, dst_hbm, e_hbm, zeros_hbm, out_hbm,
                         srcb, dstb, ebuf, None, acc, esem, gsem, ssem, isem)

    return sc_agg


_sc_agg = functools.lru_cache(maxsize=None)(_make_sc_agg)


NG_NODE = N_NODES // BN


def _edge_proj_body(ea_ref, We_ref, be_ref, e_ref):
    e_ref[...] = jnp.dot(ea_ref[...], We_ref[...],
                         preferred_element_type=jnp.float32) + be_ref[...]


def _edge_proj(ea_bf, We, be):
    dout = We.shape[1]
    grid = N_EDGES // EB
    return pl.pallas_call(
        _edge_proj_body,
        grid=(grid,),
        in_specs=[
            pl.BlockSpec((EB, D_EDGE), lambda i: (i, 0)),
            pl.BlockSpec((D_EDGE, dout), lambda i: (0, 0)),
            pl.BlockSpec((1, dout), lambda i: (0, 0)),
        ],
        out_specs=pl.BlockSpec((EB, dout), lambda i: (i, 0)),
        out_shape=jax.ShapeDtypeStruct((N_EDGES, dout), jnp.float32),
    )(ea_bf, We.astype(jnp.bfloat16), be)


def _bn_scale_shift(sa_ref, ssa_ref, g_ref, b_ref):
    mean = sa_ref[0:1, :] * (1.0 / N_NODES)
    var = ssa_ref[0:1, :] * (1.0 / N_NODES) - mean * mean
    inv = lax.rsqrt(var + BN_EPS)
    scale = inv * g_ref[...]
    shift = b_ref[...] - mean * scale
    return scale, shift


def _mlp_body(x_ref, a0_ref, a1_ref, W1_ref, b1_ref, W2_ref, b2_ref, hs_ref,
              sa_ref, ssa_ref):
    i = pl.program_id(0)

    @pl.when(i == 0)
    def _():
        sa_ref[...] = jnp.zeros_like(sa_ref)
        ssa_ref[...] = jnp.zeros_like(ssa_ref)

    u = x_ref[...] + a0_ref[...] + a1_ref[...]
    t = jnp.maximum(
        jnp.dot(u, W1_ref[...], preferred_element_type=jnp.float32)
        + b1_ref[...], 0.0)
    h = jnp.dot(t, W2_ref[...], preferred_element_type=jnp.float32) + b2_ref[...]
    hs_ref[pl.ds(i * BN, BN), :] = h
    sa_ref[...] += jnp.broadcast_to(jnp.sum(h, 0, keepdims=True), sa_ref.shape)
    ssa_ref[...] += jnp.broadcast_to(jnp.sum(h * h, 0, keepdims=True),
                                     ssa_ref.shape)


def _mlp_bn_body(x_ref, a0_ref, a1_ref, W1_ref, b1_ref, W2_ref, b2_ref,
                 g_ref, bt_ref, o_ref, hs_ref, sa_ref, ssa_ref):
    i = pl.program_id(0)

    @pl.when(i < NG_NODE)
    def _():
        _mlp_body(x_ref, a0_ref, a1_ref, W1_ref, b1_ref, W2_ref, b2_ref,
                  hs_ref, sa_ref, ssa_ref)

    @pl.when(i >= NG_NODE)
    def _():
        j = i - NG_NODE
        scale, shift = _bn_scale_shift(sa_ref, ssa_ref, g_ref, bt_ref)
        hb = hs_ref[pl.ds(j * BN, BN), :]
        o_ref[...] = jnp.maximum(hb * scale + shift, 0.0)


def _mlp_bn(x, a0, a1, W1, b1, W2, b2, g, bt):
    din = x.shape[1]
    dmid = W1.shape[1]
    dout = W2.shape[1]
    blk = lambda i: (jnp.where(i < NG_NODE, i, 0), 0)
    cst = lambda i: (0, 0)
    return pl.pallas_call(
        _mlp_bn_body,
        grid=(2 * NG_NODE,),
        in_specs=[
            pl.BlockSpec((BN, din), blk),
            pl.BlockSpec((BN, din), blk),
            pl.BlockSpec((BN, din), blk),
            pl.BlockSpec((din, dmid), cst),
            pl.BlockSpec((1, dmid), cst),
            pl.BlockSpec((dmid, dout), cst),
            pl.BlockSpec((1, dout), cst),
            pl.BlockSpec((1, dout), cst),
            pl.BlockSpec((1, dout), cst),
        ],
        out_specs=pl.BlockSpec((BN, dout),
                               lambda i: (jnp.where(i < NG_NODE, 0,
                                                    i - NG_NODE), 0)),
        out_shape=jax.ShapeDtypeStruct((N_NODES, dout), jnp.float32),
        scratch_shapes=[
            pltpu.VMEM((N_NODES, dout), jnp.float32),
            pltpu.VMEM((8, dout), jnp.float32),
            pltpu.VMEM((8, dout), jnp.float32),
        ],
    )(x, a0, a1, W1, b1, W2, b2, g, bt)


def _mlp_bn_pool_head_body(x_ref, a0_ref, a1_ref, W1_ref, b1_ref, W2_ref,
                           b2_ref, g_ref, bt_ref, batch_ref, gf_ref, Wa_ref,
                           Wb_ref, bf1_ref, Wf2_ref, bf2_ref, o_ref,
                           hs_ref, sa_ref, ssa_ref, ps_ref, cnt_ref):
    i = pl.program_id(0)

    @pl.when(i == 0)
    def _():
        ps_ref[...] = jnp.zeros_like(ps_ref)
        cnt_ref[...] = jnp.zeros_like(cnt_ref)

    @pl.when(i < NG_NODE)
    def _():
        _mlp_body(x_ref, a0_ref, a1_ref, W1_ref, b1_ref, W2_ref, b2_ref,
                  hs_ref, sa_ref, ssa_ref)

    @pl.when(i >= NG_NODE)
    def _():
        j = i - NG_NODE
        scale, shift = _bn_scale_shift(sa_ref, ssa_ref, g_ref, bt_ref)
        hb = hs_ref[pl.ds(j * BN, BN), :]
        t = jnp.maximum(hb * scale + shift, 0.0)
        bt_blk = batch_ref[0, 0, :]
        onehot = (lax.broadcasted_iota(jnp.int32, (N_GRAPHS, BN), 0)
                  == bt_blk[None, :]).astype(jnp.float32)
        ps_ref[...] += jnp.dot(onehot, t, preferred_element_type=jnp.float32)
        cnt_ref[...] += jnp.broadcast_to(jnp.sum(onehot, 1)[:, None],
                                         cnt_ref.shape)

    @pl.when(i == 2 * NG_NODE - 1)
    def _():
        cnt = jnp.maximum(cnt_ref[:, 0:1], 1.0)
        pooled = ps_ref[...] / cnt
        z = (jnp.dot(pooled, Wa_ref[...], preferred_element_type=jnp.float32)
             + jnp.dot(gf_ref[...], Wb_ref[...],
                       preferred_element_type=jnp.float32)
             + bf1_ref[...])
        z = jnp.maximum(z, 0.0)
        o_ref[...] = jnp.dot(z, Wf2_ref[...],
                             preferred_element_type=jnp.float32) + bf2_ref[...]


def _mlp_bn_pool_head(x, a0, a1, W1, b1, W2, b2, g, bt, batch3, gf, Wa, Wb,
                      bf1, Wf2, bf2):
    din = x.shape[1]
    dmid = W1.shape[1]
    dout = W2.shape[1]
    blk = lambda i: (jnp.where(i < NG_NODE, i, 0), 0)
    cst = lambda i: (0, 0)
    return pl.pallas_call(
        _mlp_bn_pool_head_body,
        grid=(2 * NG_NODE,),
        in_specs=[
            pl.BlockSpec((BN, din), blk),
            pl.BlockSpec((BN, din), blk),
            pl.BlockSpec((BN, din), blk),
            pl.BlockSpec((din, dmid), cst),
            pl.BlockSpec((1, dmid), cst),
            pl.BlockSpec((dmid, dout), cst),
            pl.BlockSpec((1, dout), cst),
            pl.BlockSpec((1, dout), cst),
            pl.BlockSpec((1, dout), cst),
            pl.BlockSpec((1, 1, BN),
                         lambda i: (jnp.where(i < NG_NODE, 0, i - NG_NODE),
                                    0, 0)),
            pl.BlockSpec((N_GRAPHS, D_GLOBAL), cst),
            pl.BlockSpec((dout, 128), cst),
            pl.BlockSpec((D_GLOBAL, 128), cst),
            pl.BlockSpec((1, 128), cst),
            pl.BlockSpec((128, 1), cst),
            pl.BlockSpec((1, 1), cst),
        ],
        out_specs=pl.BlockSpec((N_GRAPHS, 1), cst),
        out_shape=jax.ShapeDtypeStruct((N_GRAPHS, 1), jnp.float32),
        scratch_shapes=[
            pltpu.VMEM((N_NODES, dout), jnp.float32),
            pltpu.VMEM((8, dout), jnp.float32),
            pltpu.VMEM((8, dout), jnp.float32),
            pltpu.VMEM((N_GRAPHS, dout), jnp.float32),
            pltpu.VMEM((N_GRAPHS, 8), jnp.float32),
        ],
    )(x, a0, a1, W1, b1, W2, b2, g, bt, batch3, gf, Wa, Wb, bf1, Wf2, bf2)


def kernel(x, edge_index, edge_attr, batch, global_feat, We1, be1, W11, b11,
           W12, b12, g1, bt1, We2, be2, W21, b21, W22, b22, g2, bt2, Wf1,
           bf1, Wf2, bf2):
    src = edge_index[0].reshape(NW * NCH, C)
    dst = edge_index[1].reshape(NW * NCH, C)
    batch3 = batch.reshape(N_NODES // BN, 1, BN)
    zeros128 = jnp.zeros((N_NODES, 128), jnp.float32)
    zeros64 = jnp.zeros((N_NODES, 64), jnp.float32)

    ea_bf = edge_attr.astype(jnp.bfloat16)
    e1 = _edge_proj(ea_bf, We1, be1.reshape(1, -1))
    agg1 = _sc_agg(128, False)(x, src, dst, e1, zeros128)
    e2 = _edge_proj(ea_bf, We2, be2.reshape(1, -1))

    hn1 = _mlp_bn(x, agg1[0], agg1[1], W11, b11.reshape(1, -1),
                  W12, b12.reshape(1, -1), g1.reshape(1, -1),
                  bt1.reshape(1, -1))

    agg2 = _sc_agg(64, False)(hn1, src, dst, e2, zeros64)
    out = _mlp_bn_pool_head(hn1, agg2[0], agg2[1], W21, b21.reshape(1, -1),
                            W22, b22.reshape(1, -1), g2.reshape(1, -1),
                            bt2.reshape(1, -1), batch3, global_feat,
                            Wf1[:128], Wf1[128:], bf1.reshape(1, -1),
                            Wf2, bf2.reshape(1, -1))
    return out.reshape(N_GRAPHS)


# speedup vs baseline: 1.0315x; 1.0315x over previous
import functools

import jax
import jax.numpy as jnp
from jax import lax
from jax.experimental import pallas as pl
from jax.experimental.pallas import tpu as pltpu
from jax.experimental.pallas import tpu_sc as plsc

N_NODES = 10000
N_EDGES = 320000
N_GRAPHS = 256
D_NODE = 128
D_EDGE = 16
D_GLOBAL = 177
BN_EPS = 1e-5

NC = 2
NS = 16
NW = NC * NS
EPW = N_EDGES // NW
C = 80
NCH = EPW // C
ZR = 624
ZTAIL = N_NODES - NS * ZR

EB = 4000
BN = 1000


def _make_sc_agg(D, packed):
    mesh = plsc.VectorSubcoreMesh(core_axis_name="c", subcore_axis_name="s",
                                  num_cores=NC, num_subcores=NS)

    NB = 4 if D == 128 else 5
    NG = NCH // NB
    TAIL = NCH - NG * NB
    scratch_types = [
            pltpu.VMEM((NB, C), jnp.int32),
            pltpu.VMEM((NB, C), jnp.int32),
            pltpu.VMEM((NB, C // 2, 128) if packed else (NB, C, D),
                       jnp.float32),
            pltpu.VMEM_SHARED((N_NODES, D), jnp.float32),
            pltpu.SemaphoreType.DMA,
            pltpu.SemaphoreType.DMA,
            pltpu.SemaphoreType.DMA,
            pltpu.SemaphoreType.DMA,
        ]
    if packed:
        scratch_types.insert(3, pltpu.VMEM((NB, C, D), jnp.float32))

    def _body_common(x_hbm, src_hbm, dst_hbm, e_hbm, zeros_hbm, out_hbm,
                     srcb, dstb, ebuf, xbuf, acc, esem, gsem, ssem, isem):
        c = lax.axis_index("c")
        s = lax.axis_index("s")
        w = c * NS + s

        pltpu.sync_copy(zeros_hbm.at[pl.ds(s * ZR, ZR)],
                        acc.at[pl.ds(s * ZR, ZR)])

        @pl.when(s == NS - 1)
        def _():
            pltpu.sync_copy(zeros_hbm.at[pl.ds(NS * ZR, ZTAIL)],
                            acc.at[pl.ds(NS * ZR, ZTAIL)])
        plsc.subcore_barrier()

        def run_chunks(j0, n):
            idescs = [
                pltpu.async_copy(src_hbm.at[pl.ds(w * NCH + j0, n)],
                                 srcb.at[pl.ds(0, n)], isem),
                pltpu.async_copy(dst_hbm.at[pl.ds(w * NCH + j0, n)],
                                 dstb.at[pl.ds(0, n)], isem),
            ]
            edescs = []
            for b in range(n):
                base = w * EPW + (j0 + b) * C
                if packed:
                    edescs.append(pltpu.async_copy(
                        e_hbm.at[pl.ds(base // 2, C // 2)], ebuf.at[b], esem))
                else:
                    edescs.append(pltpu.async_copy(
                        e_hbm.at[pl.ds(base, C)], ebuf.at[b], esem))
            idescs[0].wait()
            idescs[1].wait()
            gdescs = []
            for b in range(n):
                if packed:
                    gdescs.append(pltpu.async_copy(
                        x_hbm.at[srcb.at[b]], xbuf.at[b], gsem))
                else:
                    edescs[b].wait()
                    gdescs.append(pltpu.async_copy(
                        x_hbm.at[srcb.at[b]], ebuf.at[b], gsem, add=True))
            sdescs = []
            for b in range(n):
                gdescs[b].wait()
                if packed:
                    edescs[b].wait()

                    def rows2(rr, carry2, b=b):
                        for half in range(2):
                            for k in range(D // 16):
                                xs = (b, 2 * rr + half, pl.ds(k * 16, 16))
                                es = (b, rr, pl.ds(half * 64 + k * 16, 16))
                                xbuf[xs] = jnp.maximum(xbuf[xs] + ebuf[es],
                                                       0.0)
                        return carry2
                    lax.fori_loop(0, C // 2, rows2, 0)
                    src_buf = xbuf
                else:
                    rpi = 256 // D

                    def rows(r, carry2, b=b):
                        for u in range(rpi):
                            for k in range(D // 16):
                                sl = (b, r * rpi + u, pl.ds(k * 16, 16))
                                ebuf[sl] = jnp.maximum(ebuf[sl], 0.0)
                        return carry2
                    lax.fori_loop(0, C // rpi, rows, 0)
                    src_buf = ebuf
                sdescs.append(pltpu.async_copy(
                    src_buf.at[b], acc.at[dstb.at[b]], ssem, add=True))
            for b in range(n):
                sdescs[b].wait()

        def group(g, carry):
            run_chunks(g * NB, NB)
            return carry
        lax.fori_loop(0, NG, group, 0)
        if TAIL:
            run_chunks(NG * NB, TAIL)

        plsc.subcore_barrier()
        pltpu.sync_copy(acc.at[pl.ds(s * ZR, ZR)],
                        out_hbm.at[c, pl.ds(s * ZR, ZR)])

        @pl.when(s == NS - 1)
        def _():
            pltpu.sync_copy(acc.at[pl.ds(NS * ZR, ZTAIL)],
                            out_hbm.at[c, pl.ds(NS * ZR, ZTAIL)])

    deco = functools.partial(
        pl.kernel,
        out_type=jax.ShapeDtypeStruct((NC, N_NODES, D), jnp.float32),
        mesh=mesh,
        scratch_types=scratch_types,
        compiler_params=pltpu.CompilerParams(use_tc_tiling_on_sc=False),
    )
    if packed:
        @deco
        def sc_agg(x_hbm, src_hbm, dst_hbm, e_hbm, zeros_hbm, out_hbm,
                   srcb, dstb, ebuf, xbuf, acc, esem, gsem, ssem, isem):
            _body_common(x_hbm, src_hbm, dst_hbm, e_hbm, zeros_hbm, out_hbm,
                         srcb, dstb, ebuf, xbuf, acc, esem, gsem, ssem, isem)
    else:
        @deco
        def sc_agg(x_hbm, src_hbm, dst_hbm, e_hbm, zeros_hbm, out_hbm,
                   srcb, dstb, ebuf, acc, esem, gsem, ssem, isem):
            _body_common(x_hbm, src_hbm, dst_hbm, e_hbm, zeros_hbm, out_hbm,
                         srcb, dstb, ebuf, None, acc, esem, gsem, ssem, isem)

    return sc_agg


_sc_agg = functools.lru_cache(maxsize=None)(_make_sc_agg)


NG_NODE = N_NODES // BN


def _edge_feat_body(ea_ref, We1_ref, be1_ref, We2_ref, be2_ref, e1_ref, e2_ref):
    ea = ea_ref[...]
    e1_ref[...] = jnp.dot(ea, We1_ref[...],
                          preferred_element_type=jnp.float32) + be1_ref[...]
    e2_ref[...] = jnp.dot(ea, We2_ref[...],
                          preferred_element_type=jnp.float32) + be2_ref[...]


def _edge_feats(edge_attr, We1, be1, We2, be2):
    grid = N_EDGES // EB
    return pl.pallas_call(
        _edge_feat_body,
        grid=(grid,),
        in_specs=[
            pl.BlockSpec((EB, D_EDGE), lambda i: (i, 0)),
            pl.BlockSpec((D_EDGE, D_NODE), lambda i: (0, 0)),
            pl.BlockSpec((1, D_NODE), lambda i: (0, 0)),
            pl.BlockSpec((D_EDGE, 64), lambda i: (0, 0)),
            pl.BlockSpec((1, 64), lambda i: (0, 0)),
        ],
        out_specs=[
            pl.BlockSpec((EB, D_NODE), lambda i: (i, 0)),
            pl.BlockSpec((EB, 64), lambda i: (i, 0)),
        ],
        out_shape=[
            jax.ShapeDtypeStruct((N_EDGES, D_NODE), jnp.float32),
            jax.ShapeDtypeStruct((N_EDGES, 64), jnp.float32),
        ],
    )(edge_attr.astype(jnp.bfloat16), We1.astype(jnp.bfloat16), be1,
      We2.astype(jnp.bfloat16), be2)


def _bn_scale_shift(sa_ref, ssa_ref, g_ref, b_ref):
    mean = sa_ref[0:1, :] * (1.0 / N_NODES)
    var = ssa_ref[0:1, :] * (1.0 / N_NODES) - mean * mean
    inv = lax.rsqrt(var + BN_EPS)
    scale = inv * g_ref[...]
    shift = b_ref[...] - mean * scale
    return scale, shift


def _mlp_body(x_ref, a0_ref, a1_ref, W1_ref, b1_ref, W2_ref, b2_ref, hs_ref,
              sa_ref, ssa_ref):
    i = pl.program_id(0)

    @pl.when(i == 0)
    def _():
        sa_ref[...] = jnp.zeros_like(sa_ref)
        ssa_ref[...] = jnp.zeros_like(ssa_ref)

    u = x_ref[...] + a0_ref[...] + a1_ref[...]
    t = jnp.maximum(
        jnp.dot(u, W1_ref[...], preferred_element_type=jnp.float32)
        + b1_ref[...], 0.0)
    h = jnp.dot(t, W2_ref[...], preferred_element_type=jnp.float32) + b2_ref[...]
    hs_ref[pl.ds(i * BN, BN), :] = h
    sa_ref[...] += jnp.broadcast_to(jnp.sum(h, 0, keepdims=True), sa_ref.shape)
    ssa_ref[...] += jnp.broadcast_to(jnp.sum(h * h, 0, keepdims=True),
                                     ssa_ref.shape)


def _mlp_bn_body(x_ref, a0_ref, a1_ref, W1_ref, b1_ref, W2_ref, b2_ref,
                 g_ref, bt_ref, o_ref, hs_ref, sa_ref, ssa_ref):
    i = pl.program_id(0)

    @pl.when(i < NG_NODE)
    def _():
        _mlp_body(x_ref, a0_ref, a1_ref, W1_ref, b1_ref, W2_ref, b2_ref,
                  hs_ref, sa_ref, ssa_ref)

    @pl.when(i >= NG_NODE)
    def _():
        j = i - NG_NODE
        scale, shift = _bn_scale_shift(sa_ref, ssa_ref, g_ref, bt_ref)
        hb = hs_ref[pl.ds(j * BN, BN), :]
        o_ref[...] = jnp.maximum(hb * scale + shift, 0.0)


def _mlp_bn(x, a0, a1, W1, b1, W2, b2, g, bt):
    din = x.shape[1]
    dmid = W1.shape[1]
    dout = W2.shape[1]
    blk = lambda i: (jnp.where(i < NG_NODE, i, 0), 0)
    cst = lambda i: (0, 0)
    return pl.pallas_call(
        _mlp_bn_body,
        grid=(2 * NG_NODE,),
        in_specs=[
            pl.BlockSpec((BN, din), blk),
            pl.BlockSpec((BN, din), blk),
            pl.BlockSpec((BN, din), blk),
            pl.BlockSpec((din, dmid), cst),
            pl.BlockSpec((1, dmid), cst),
            pl.BlockSpec((dmid, dout), cst),
            pl.BlockSpec((1, dout), cst),
            pl.BlockSpec((1, dout), cst),
            pl.BlockSpec((1, dout), cst),
        ],
        out_specs=pl.BlockSpec((BN, dout),
                               lambda i: (jnp.where(i < NG_NODE, 0,
                                                    i - NG_NODE), 0)),
        out_shape=jax.ShapeDtypeStruct((N_NODES, dout), jnp.float32),
        scratch_shapes=[
            pltpu.VMEM((N_NODES, dout), jnp.float32),
            pltpu.VMEM((8, dout), jnp.float32),
            pltpu.VMEM((8, dout), jnp.float32),
        ],
    )(x, a0, a1, W1, b1, W2, b2, g, bt)


def _mlp_bn_pool_head_body(x_ref, a0_ref, a1_ref, W1_ref, b1_ref, W2_ref,
                           b2_ref, g_ref, bt_ref, batch_ref, gf_ref, Wa_ref,
                           Wb_ref, bf1_ref, Wf2_ref, bf2_ref, o_ref,
                           hs_ref, sa_ref, ssa_ref, ps_ref, cnt_ref):
    i = pl.program_id(0)

    @pl.when(i == 0)
    def _():
        ps_ref[...] = jnp.zeros_like(ps_ref)
        cnt_ref[...] = jnp.zeros_like(cnt_ref)

    @pl.when(i < NG_NODE)
    def _():
        _mlp_body(x_ref, a0_ref, a1_ref, W1_ref, b1_ref, W2_ref, b2_ref,
                  hs_ref, sa_ref, ssa_ref)

    @pl.when(i >= NG_NODE)
    def _():
        j = i - NG_NODE
        scale, shift = _bn_scale_shift(sa_ref, ssa_ref, g_ref, bt_ref)
        hb = hs_ref[pl.ds(j * BN, BN), :]
        t = jnp.maximum(hb * scale + shift, 0.0)
        bt_blk = batch_ref[0, 0, :]
        onehot = (lax.broadcasted_iota(jnp.int32, (N_GRAPHS, BN), 0)
                  == bt_blk[None, :]).astype(jnp.float32)
        ps_ref[...] += jnp.dot(onehot, t, preferred_element_type=jnp.float32)
        cnt_ref[...] += jnp.broadcast_to(jnp.sum(onehot, 1)[:, None],
                                         cnt_ref.shape)

    @pl.when(i == 2 * NG_NODE - 1)
    def _():
        cnt = jnp.maximum(cnt_ref[:, 0:1], 1.0)
        pooled = ps_ref[...] / cnt
        z = (jnp.dot(pooled, Wa_ref[...], preferred_element_type=jnp.float32)
             + jnp.dot(gf_ref[...], Wb_ref[...],
                       preferred_element_type=jnp.float32)
             + bf1_ref[...])
        z = jnp.maximum(z, 0.0)
        o_ref[...] = jnp.dot(z, Wf2_ref[...],
                             preferred_element_type=jnp.float32) + bf2_ref[...]


def _mlp_bn_pool_head(x, a0, a1, W1, b1, W2, b2, g, bt, batch3, gf, Wa, Wb,
                      bf1, Wf2, bf2):
    din = x.shape[1]
    dmid = W1.shape[1]
    dout = W2.shape[1]
    blk = lambda i: (jnp.where(i < NG_NODE, i, 0), 0)
    cst = lambda i: (0, 0)
    return pl.pallas_call(
        _mlp_bn_pool_head_body,
        grid=(2 * NG_NODE,),
        in_specs=[
            pl.BlockSpec((BN, din), blk),
            pl.BlockSpec((BN, din), blk),
            pl.BlockSpec((BN, din), blk),
            pl.BlockSpec((din, dmid), cst),
            pl.BlockSpec((1, dmid), cst),
            pl.BlockSpec((dmid, dout), cst),
            pl.BlockSpec((1, dout), cst),
            pl.BlockSpec((1, dout), cst),
            pl.BlockSpec((1, dout), cst),
            pl.BlockSpec((1, 1, BN),
                         lambda i: (jnp.where(i < NG_NODE, 0, i - NG_NODE),
                                    0, 0)),
            pl.BlockSpec((N_GRAPHS, D_GLOBAL), cst),
            pl.BlockSpec((dout, 128), cst),
            pl.BlockSpec((D_GLOBAL, 128), cst),
            pl.BlockSpec((1, 128), cst),
            pl.BlockSpec((128, 1), cst),
            pl.BlockSpec((1, 1), cst),
        ],
        out_specs=pl.BlockSpec((N_GRAPHS, 1), cst),
        out_shape=jax.ShapeDtypeStruct((N_GRAPHS, 1), jnp.float32),
        scratch_shapes=[
            pltpu.VMEM((N_NODES, dout), jnp.float32),
            pltpu.VMEM((8, dout), jnp.float32),
            pltpu.VMEM((8, dout), jnp.float32),
            pltpu.VMEM((N_GRAPHS, dout), jnp.float32),
            pltpu.VMEM((N_GRAPHS, 8), jnp.float32),
        ],
    )(x, a0, a1, W1, b1, W2, b2, g, bt, batch3, gf, Wa, Wb, bf1, Wf2, bf2)


def kernel(x, edge_index, edge_attr, batch, global_feat, We1, be1, W11, b11,
           W12, b12, g1, bt1, We2, be2, W21, b21, W22, b22, g2, bt2, Wf1,
           bf1, Wf2, bf2):
    src = edge_index[0].reshape(NW * NCH, C)
    dst = edge_index[1].reshape(NW * NCH, C)
    batch3 = batch.reshape(N_NODES // BN, 1, BN)
    zeros128 = jnp.zeros((N_NODES, 128), jnp.float32)
    zeros64 = jnp.zeros((N_NODES, 64), jnp.float32)

    e1, e2 = _edge_feats(edge_attr, We1, be1.reshape(1, -1),
                         We2, be2.reshape(1, -1))
    agg1 = _sc_agg(128, False)(x, src, dst, e1, zeros128)

    hn1 = _mlp_bn(x, agg1[0], agg1[1], W11, b11.reshape(1, -1),
                  W12, b12.reshape(1, -1), g1.reshape(1, -1),
                  bt1.reshape(1, -1))

    agg2 = _sc_agg(64, False)(hn1, src, dst, e2, zeros64)
    out = _mlp_bn_pool_head(hn1, agg2[0], agg2[1], W21, b21.reshape(1, -1),
                            W22, b22.reshape(1, -1), g2.reshape(1, -1),
                            bt2.reshape(1, -1), batch3, global_feat,
                            Wf1[:128], Wf1[128:], bf1.reshape(1, -1),
                            Wf2, bf2.reshape(1, -1))
    return out.reshape(N_GRAPHS)


# speedup vs baseline: 1.0501x; 1.0180x over previous
import functools

import jax
import jax.numpy as jnp
from jax import lax
from jax.experimental import pallas as pl
from jax.experimental.pallas import tpu as pltpu
from jax.experimental.pallas import tpu_sc as plsc

N_NODES = 10000
N_EDGES = 320000
N_GRAPHS = 256
D_NODE = 128
D_EDGE = 16
D_GLOBAL = 177
BN_EPS = 1e-5

NC = 2
NS = 16
NW = NC * NS
EPW = N_EDGES // NW
C = 80
NCH = EPW // C
ZR = 624
ZTAIL = N_NODES - NS * ZR

EB = 8000
BN = 1000


def _make_sc_agg(D, packed):
    mesh = plsc.VectorSubcoreMesh(core_axis_name="c", subcore_axis_name="s",
                                  num_cores=NC, num_subcores=NS)

    NB = 4 if D == 128 else 5
    NG = NCH // NB
    TAIL = NCH - NG * NB
    scratch_types = [
            pltpu.VMEM((NB, C), jnp.int32),
            pltpu.VMEM((NB, C), jnp.int32),
            pltpu.VMEM((NB, C // 2, 128) if packed else (NB, C, D),
                       jnp.float32),
            pltpu.VMEM_SHARED((N_NODES, D), jnp.float32),
            pltpu.SemaphoreType.DMA,
            pltpu.SemaphoreType.DMA,
            pltpu.SemaphoreType.DMA,
            pltpu.SemaphoreType.DMA,
        ]
    if packed:
        scratch_types.insert(3, pltpu.VMEM((NB, C, D), jnp.float32))

    def _body_common(x_hbm, src_hbm, dst_hbm, e_hbm, zeros_hbm, out_hbm,
                     srcb, dstb, ebuf, xbuf, acc, esem, gsem, ssem, isem):
        c = lax.axis_index("c")
        s = lax.axis_index("s")
        w = c * NS + s

        pltpu.sync_copy(zeros_hbm.at[pl.ds(s * ZR, ZR)],
                        acc.at[pl.ds(s * ZR, ZR)])

        @pl.when(s == NS - 1)
        def _():
            pltpu.sync_copy(zeros_hbm.at[pl.ds(NS * ZR, ZTAIL)],
                            acc.at[pl.ds(NS * ZR, ZTAIL)])
        plsc.subcore_barrier()

        def run_chunks(j0, n):
            idescs = [
                pltpu.async_copy(src_hbm.at[pl.ds(w * NCH + j0, n)],
                                 srcb.at[pl.ds(0, n)], isem),
                pltpu.async_copy(dst_hbm.at[pl.ds(w * NCH + j0, n)],
                                 dstb.at[pl.ds(0, n)], isem),
            ]
            edescs = []
            for b in range(n):
                base = w * EPW + (j0 + b) * C
                if packed:
                    edescs.append(pltpu.async_copy(
                        e_hbm.at[pl.ds(base // 2, C // 2)], ebuf.at[b], esem))
                else:
                    edescs.append(pltpu.async_copy(
                        e_hbm.at[pl.ds(base, C)], ebuf.at[b], esem))
            idescs[0].wait()
            idescs[1].wait()
            gdescs = []
            for b in range(n):
                if packed:
                    gdescs.append(pltpu.async_copy(
                        x_hbm.at[srcb.at[b]], xbuf.at[b], gsem))
                else:
                    edescs[b].wait()
                    gdescs.append(pltpu.async_copy(
                        x_hbm.at[srcb.at[b]], ebuf.at[b], gsem, add=True))
            sdescs = []
            for b in range(n):
                gdescs[b].wait()
                if packed:
                    edescs[b].wait()

                    def rows2(rr, carry2, b=b):
                        for half in range(2):
                            for k in range(D // 16):
                                xs = (b, 2 * rr + half, pl.ds(k * 16, 16))
                                es = (b, rr, pl.ds(half * 64 + k * 16, 16))
                                xbuf[xs] = jnp.maximum(xbuf[xs] + ebuf[es],
                                                       0.0)
                        return carry2
                    lax.fori_loop(0, C // 2, rows2, 0)
                    src_buf = xbuf
                else:
                    rpi = 256 // D

                    def rows(r, carry2, b=b):
                        for u in range(rpi):
                            for k in range(D // 16):
                                sl = (b, r * rpi + u, pl.ds(k * 16, 16))
                                ebuf[sl] = jnp.maximum(ebuf[sl], 0.0)
                        return carry2
                    lax.fori_loop(0, C // rpi, rows, 0)
                    src_buf = ebuf
                sdescs.append(pltpu.async_copy(
                    src_buf.at[b], acc.at[dstb.at[b]], ssem, add=True))
            for b in range(n):
                sdescs[b].wait()

        def group(g, carry):
            run_chunks(g * NB, NB)
            return carry
        lax.fori_loop(0, NG, group, 0)
        if TAIL:
            run_chunks(NG * NB, TAIL)

        plsc.subcore_barrier()
        pltpu.sync_copy(acc.at[pl.ds(s * ZR, ZR)],
                        out_hbm.at[c, pl.ds(s * ZR, ZR)])

        @pl.when(s == NS - 1)
        def _():
            pltpu.sync_copy(acc.at[pl.ds(NS * ZR, ZTAIL)],
                            out_hbm.at[c, pl.ds(NS * ZR, ZTAIL)])

    deco = functools.partial(
        pl.kernel,
        out_type=jax.ShapeDtypeStruct((NC, N_NODES, D), jnp.float32),
        mesh=mesh,
        scratch_types=scratch_types,
        compiler_params=pltpu.CompilerParams(use_tc_tiling_on_sc=False),
    )
    if packed:
        @deco
        def sc_agg(x_hbm, src_hbm, dst_hbm, e_hbm, zeros_hbm, out_hbm,
                   srcb, dstb, ebuf, xbuf, acc, esem, gsem, ssem, isem):
            _body_common(x_hbm, src_hbm, dst_hbm, e_hbm, zeros_hbm, out_hbm,
                         srcb, dstb, ebuf, xbuf, acc, esem, gsem, ssem, isem)
    else:
        @deco
        def sc_agg(x_hbm, src_hbm, dst_hbm, e_hbm, zeros_hbm, out_hbm,
                   srcb, dstb, ebuf, acc, esem, gsem, ssem, isem):
            _body_common(x_hbm, src_hbm, dst_hbm, e_hbm, zeros_hbm, out_hbm,
                         srcb, dstb, ebuf, None, acc, esem, gsem, ssem, isem)

    return sc_agg


_sc_agg = functools.lru_cache(maxsize=None)(_make_sc_agg)


NG_NODE = N_NODES // BN


def _edge_feat_body(ea_ref, We1_ref, be1_ref, We2_ref, be2_ref, e1_ref, e2_ref):
    ea = ea_ref[...]
    e1_ref[...] = jnp.dot(ea, We1_ref[...],
                          preferred_element_type=jnp.float32) + be1_ref[...]
    e2_ref[...] = jnp.dot(ea, We2_ref[...],
                          preferred_element_type=jnp.float32) + be2_ref[...]


def _edge_feats(edge_attr, We1, be1, We2, be2):
    grid = N_EDGES // EB
    return pl.pallas_call(
        _edge_feat_body,
        grid=(grid,),
        in_specs=[
            pl.BlockSpec((EB, D_EDGE), lambda i: (i, 0)),
            pl.BlockSpec((D_EDGE, D_NODE), lambda i: (0, 0)),
            pl.BlockSpec((1, D_NODE), lambda i: (0, 0)),
            pl.BlockSpec((D_EDGE, 64), lambda i: (0, 0)),
            pl.BlockSpec((1, 64), lambda i: (0, 0)),
        ],
        out_specs=[
            pl.BlockSpec((EB, D_NODE), lambda i: (i, 0)),
            pl.BlockSpec((EB, 64), lambda i: (i, 0)),
        ],
        out_shape=[
            jax.ShapeDtypeStruct((N_EDGES, D_NODE), jnp.float32),
            jax.ShapeDtypeStruct((N_EDGES, 64), jnp.float32),
        ],
    )(edge_attr.astype(jnp.bfloat16), We1.astype(jnp.bfloat16), be1,
      We2.astype(jnp.bfloat16), be2)


def _bn_scale_shift(sa_ref, ssa_ref, g_ref, b_ref):
    mean = sa_ref[0:1, :] * (1.0 / N_NODES)
    var = ssa_ref[0:1, :] * (1.0 / N_NODES) - mean * mean
    inv = lax.rsqrt(var + BN_EPS)
    scale = inv * g_ref[...]
    shift = b_ref[...] - mean * scale
    return scale, shift


def _mlp_body(x_ref, a0_ref, a1_ref, W1_ref, b1_ref, W2_ref, b2_ref, hs_ref,
              sa_ref, ssa_ref):
    i = pl.program_id(0)

    @pl.when(i == 0)
    def _():
        sa_ref[...] = jnp.zeros_like(sa_ref)
        ssa_ref[...] = jnp.zeros_like(ssa_ref)

    u = x_ref[...] + a0_ref[...] + a1_ref[...]
    t = jnp.maximum(
        jnp.dot(u, W1_ref[...], preferred_element_type=jnp.float32)
        + b1_ref[...], 0.0)
    h = jnp.dot(t, W2_ref[...], preferred_element_type=jnp.float32) + b2_ref[...]
    hs_ref[pl.ds(i * BN, BN), :] = h
    sa_ref[...] += jnp.broadcast_to(jnp.sum(h, 0, keepdims=True), sa_ref.shape)
    ssa_ref[...] += jnp.broadcast_to(jnp.sum(h * h, 0, keepdims=True),
                                     ssa_ref.shape)


def _mlp_bn_body(x_ref, a0_ref, a1_ref, W1_ref, b1_ref, W2_ref, b2_ref,
                 g_ref, bt_ref, o_ref, hs_ref, sa_ref, ssa_ref):
    i = pl.program_id(0)

    @pl.when(i < NG_NODE)
    def _():
        _mlp_body(x_ref, a0_ref, a1_ref, W1_ref, b1_ref, W2_ref, b2_ref,
                  hs_ref, sa_ref, ssa_ref)

    @pl.when(i >= NG_NODE)
    def _():
        j = i - NG_NODE
        scale, shift = _bn_scale_shift(sa_ref, ssa_ref, g_ref, bt_ref)
        hb = hs_ref[pl.ds(j * BN, BN), :]
        o_ref[...] = jnp.maximum(hb * scale + shift, 0.0)


def _mlp_bn(x, a0, a1, W1, b1, W2, b2, g, bt):
    din = x.shape[1]
    dmid = W1.shape[1]
    dout = W2.shape[1]
    blk = lambda i: (jnp.where(i < NG_NODE, i, 0), 0)
    cst = lambda i: (0, 0)
    return pl.pallas_call(
        _mlp_bn_body,
        grid=(2 * NG_NODE,),
        in_specs=[
            pl.BlockSpec((BN, din), blk),
            pl.BlockSpec((BN, din), blk),
            pl.BlockSpec((BN, din), blk),
            pl.BlockSpec((din, dmid), cst),
            pl.BlockSpec((1, dmid), cst),
            pl.BlockSpec((dmid, dout), cst),
            pl.BlockSpec((1, dout), cst),
            pl.BlockSpec((1, dout), cst),
            pl.BlockSpec((1, dout), cst),
        ],
        out_specs=pl.BlockSpec((BN, dout),
                               lambda i: (jnp.where(i < NG_NODE, 0,
                                                    i - NG_NODE), 0)),
        out_shape=jax.ShapeDtypeStruct((N_NODES, dout), jnp.float32),
        scratch_shapes=[
            pltpu.VMEM((N_NODES, dout), jnp.float32),
            pltpu.VMEM((8, dout), jnp.float32),
            pltpu.VMEM((8, dout), jnp.float32),
        ],
    )(x, a0, a1, W1, b1, W2, b2, g, bt)


def _mlp_bn_pool_head_body(x_ref, a0_ref, a1_ref, W1_ref, b1_ref, W2_ref,
                           b2_ref, g_ref, bt_ref, batch_ref, gf_ref, Wa_ref,
                           Wb_ref, bf1_ref, Wf2_ref, bf2_ref, o_ref,
                           hs_ref, sa_ref, ssa_ref, ps_ref, cnt_ref):
    i = pl.program_id(0)

    @pl.when(i == 0)
    def _():
        ps_ref[...] = jnp.zeros_like(ps_ref)
        cnt_ref[...] = jnp.zeros_like(cnt_ref)

    @pl.when(i < NG_NODE)
    def _():
        _mlp_body(x_ref, a0_ref, a1_ref, W1_ref, b1_ref, W2_ref, b2_ref,
                  hs_ref, sa_ref, ssa_ref)

    @pl.when(i >= NG_NODE)
    def _():
        j = i - NG_NODE
        scale, shift = _bn_scale_shift(sa_ref, ssa_ref, g_ref, bt_ref)
        hb = hs_ref[pl.ds(j * BN, BN), :]
        t = jnp.maximum(hb * scale + shift, 0.0)
        bt_blk = batch_ref[0, 0, :]
        onehot = (lax.broadcasted_iota(jnp.int32, (N_GRAPHS, BN), 0)
                  == bt_blk[None, :]).astype(jnp.float32)
        ps_ref[...] += jnp.dot(onehot, t, preferred_element_type=jnp.float32)
        cnt_ref[...] += jnp.broadcast_to(jnp.sum(onehot, 1)[:, None],
                                         cnt_ref.shape)

    @pl.when(i == 2 * NG_NODE - 1)
    def _():
        cnt = jnp.maximum(cnt_ref[:, 0:1], 1.0)
        pooled = ps_ref[...] / cnt
        z = (jnp.dot(pooled, Wa_ref[...], preferred_element_type=jnp.float32)
             + jnp.dot(gf_ref[...], Wb_ref[...],
                       preferred_element_type=jnp.float32)
             + bf1_ref[...])
        z = jnp.maximum(z, 0.0)
        o_ref[...] = jnp.dot(z, Wf2_ref[...],
                             preferred_element_type=jnp.float32) + bf2_ref[...]


def _mlp_bn_pool_head(x, a0, a1, W1, b1, W2, b2, g, bt, batch3, gf, Wa, Wb,
                      bf1, Wf2, bf2):
    din = x.shape[1]
    dmid = W1.shape[1]
    dout = W2.shape[1]
    blk = lambda i: (jnp.where(i < NG_NODE, i, 0), 0)
    cst = lambda i: (0, 0)
    return pl.pallas_call(
        _mlp_bn_pool_head_body,
        grid=(2 * NG_NODE,),
        in_specs=[
            pl.BlockSpec((BN, din), blk),
            pl.BlockSpec((BN, din), blk),
            pl.BlockSpec((BN, din), blk),
            pl.BlockSpec((din, dmid), cst),
            pl.BlockSpec((1, dmid), cst),
            pl.BlockSpec((dmid, dout), cst),
            pl.BlockSpec((1, dout), cst),
            pl.BlockSpec((1, dout), cst),
            pl.BlockSpec((1, dout), cst),
            pl.BlockSpec((1, 1, BN),
                         lambda i: (jnp.where(i < NG_NODE, 0, i - NG_NODE),
                                    0, 0)),
            pl.BlockSpec((N_GRAPHS, D_GLOBAL), cst),
            pl.BlockSpec((dout, 128), cst),
            pl.BlockSpec((D_GLOBAL, 128), cst),
            pl.BlockSpec((1, 128), cst),
            pl.BlockSpec((128, 1), cst),
            pl.BlockSpec((1, 1), cst),
        ],
        out_specs=pl.BlockSpec((N_GRAPHS, 1), cst),
        out_shape=jax.ShapeDtypeStruct((N_GRAPHS, 1), jnp.float32),
        scratch_shapes=[
            pltpu.VMEM((N_NODES, dout), jnp.float32),
            pltpu.VMEM((8, dout), jnp.float32),
            pltpu.VMEM((8, dout), jnp.float32),
            pltpu.VMEM((N_GRAPHS, dout), jnp.float32),
            pltpu.VMEM((N_GRAPHS, 8), jnp.float32),
        ],
    )(x, a0, a1, W1, b1, W2, b2, g, bt, batch3, gf, Wa, Wb, bf1, Wf2, bf2)


def kernel(x, edge_index, edge_attr, batch, global_feat, We1, be1, W11, b11,
           W12, b12, g1, bt1, We2, be2, W21, b21, W22, b22, g2, bt2, Wf1,
           bf1, Wf2, bf2):
    src = edge_index[0].reshape(NW * NCH, C)
    dst = edge_index[1].reshape(NW * NCH, C)
    batch3 = batch.reshape(N_NODES // BN, 1, BN)
    zeros128 = jnp.zeros((N_NODES, 128), jnp.float32)
    zeros64 = jnp.zeros((N_NODES, 64), jnp.float32)

    e1, e2 = _edge_feats(edge_attr, We1, be1.reshape(1, -1),
                         We2, be2.reshape(1, -1))
    agg1 = _sc_agg(128, False)(x, src, dst, e1, zeros128)

    hn1 = _mlp_bn(x, agg1[0], agg1[1], W11, b11.reshape(1, -1),
                  W12, b12.reshape(1, -1), g1.reshape(1, -1),
                  bt1.reshape(1, -1))

    agg2 = _sc_agg(64, False)(hn1, src, dst, e2, zeros64)
    out = _mlp_bn_pool_head(hn1, agg2[0], agg2[1], W21, b21.reshape(1, -1),
                            W22, b22.reshape(1, -1), g2.reshape(1, -1),
                            bt2.reshape(1, -1), batch3, global_feat,
                            Wf1[:128], Wf1[128:], bf1.reshape(1, -1),
                            Wf2, bf2.reshape(1, -1))
    return out.reshape(N_GRAPHS)


# speedup vs baseline: 1.0568x; 1.0064x over previous
import functools

import jax
import jax.numpy as jnp
from jax import lax
from jax.experimental import pallas as pl
from jax.experimental.pallas import tpu as pltpu
from jax.experimental.pallas import tpu_sc as plsc

N_NODES = 10000
N_EDGES = 320000
N_GRAPHS = 256
D_NODE = 128
D_EDGE = 16
D_GLOBAL = 177
BN_EPS = 1e-5

NC = 2
NS = 16
NW = NC * NS
EPW = N_EDGES // NW
C = 80
NCH = EPW // C
ZR = 624
ZTAIL = N_NODES - NS * ZR

EB = 16000
BN = 1000


def _make_sc_agg(D, packed):
    mesh = plsc.VectorSubcoreMesh(core_axis_name="c", subcore_axis_name="s",
                                  num_cores=NC, num_subcores=NS)

    NB = 4 if D == 128 else 5
    NG = NCH // NB
    TAIL = NCH - NG * NB
    scratch_types = [
            pltpu.VMEM((NB, C), jnp.int32),
            pltpu.VMEM((NB, C), jnp.int32),
            pltpu.VMEM((NB, C // 2, 128) if packed else (NB, C, D),
                       jnp.float32),
            pltpu.VMEM_SHARED((N_NODES, D), jnp.float32),
            pltpu.SemaphoreType.DMA,
            pltpu.SemaphoreType.DMA,
            pltpu.SemaphoreType.DMA,
            pltpu.SemaphoreType.DMA,
        ]
    if packed:
        scratch_types.insert(3, pltpu.VMEM((NB, C, D), jnp.float32))

    def _body_common(x_hbm, src_hbm, dst_hbm, e_hbm, zeros_hbm, out_hbm,
                     srcb, dstb, ebuf, xbuf, acc, esem, gsem, ssem, isem):
        c = lax.axis_index("c")
        s = lax.axis_index("s")
        w = c * NS + s

        pltpu.sync_copy(zeros_hbm.at[pl.ds(s * ZR, ZR)],
                        acc.at[pl.ds(s * ZR, ZR)])

        @pl.when(s == NS - 1)
        def _():
            pltpu.sync_copy(zeros_hbm.at[pl.ds(NS * ZR, ZTAIL)],
                            acc.at[pl.ds(NS * ZR, ZTAIL)])
        plsc.subcore_barrier()

        def run_chunks(j0, n):
            idescs = [
                pltpu.async_copy(src_hbm.at[pl.ds(w * NCH + j0, n)],
                                 srcb.at[pl.ds(0, n)], isem),
                pltpu.async_copy(dst_hbm.at[pl.ds(w * NCH + j0, n)],
                                 dstb.at[pl.ds(0, n)], isem),
            ]
            edescs = []
            for b in range(n):
                base = w * EPW + (j0 + b) * C
                if packed:
                    edescs.append(pltpu.async_copy(
                        e_hbm.at[pl.ds(base // 2, C // 2)], ebuf.at[b], esem))
                else:
                    edescs.append(pltpu.async_copy(
                        e_hbm.at[pl.ds(base, C)], ebuf.at[b], esem))
            idescs[0].wait()
            idescs[1].wait()
            gdescs = []
            for b in range(n):
                if packed:
                    gdescs.append(pltpu.async_copy(
                        x_hbm.at[srcb.at[b]], xbuf.at[b], gsem))
                else:
                    edescs[b].wait()
                    gdescs.append(pltpu.async_copy(
                        x_hbm.at[srcb.at[b]], ebuf.at[b], gsem, add=True))
            sdescs = []
            for b in range(n):
                gdescs[b].wait()
                if packed:
                    edescs[b].wait()

                    def rows2(rr, carry2, b=b):
                        for half in range(2):
                            for k in range(D // 16):
                                xs = (b, 2 * rr + half, pl.ds(k * 16, 16))
                                es = (b, rr, pl.ds(half * 64 + k * 16, 16))
                                xbuf[xs] = jnp.maximum(xbuf[xs] + ebuf[es],
                                                       0.0)
                        return carry2
                    lax.fori_loop(0, C // 2, rows2, 0)
                    src_buf = xbuf
                else:
                    rpi = 256 // D

                    def rows(r, carry2, b=b):
                        for u in range(rpi):
                            for k in range(D // 16):
                                sl = (b, r * rpi + u, pl.ds(k * 16, 16))
                                ebuf[sl] = jnp.maximum(ebuf[sl], 0.0)
                        return carry2
                    lax.fori_loop(0, C // rpi, rows, 0)
                    src_buf = ebuf
                sdescs.append(pltpu.async_copy(
                    src_buf.at[b], acc.at[dstb.at[b]], ssem, add=True))
            for b in range(n):
                sdescs[b].wait()

        def group(g, carry):
            run_chunks(g * NB, NB)
            return carry
        lax.fori_loop(0, NG, group, 0)
        if TAIL:
            run_chunks(NG * NB, TAIL)

        plsc.subcore_barrier()
        pltpu.sync_copy(acc.at[pl.ds(s * ZR, ZR)],
                        out_hbm.at[c, pl.ds(s * ZR, ZR)])

        @pl.when(s == NS - 1)
        def _():
            pltpu.sync_copy(acc.at[pl.ds(NS * ZR, ZTAIL)],
                            out_hbm.at[c, pl.ds(NS * ZR, ZTAIL)])

    deco = functools.partial(
        pl.kernel,
        out_type=jax.ShapeDtypeStruct((NC, N_NODES, D), jnp.float32),
        mesh=mesh,
        scratch_types=scratch_types,
        compiler_params=pltpu.CompilerParams(use_tc_tiling_on_sc=False),
    )
    if packed:
        @deco
        def sc_agg(x_hbm, src_hbm, dst_hbm, e_hbm, zeros_hbm, out_hbm,
                   srcb, dstb, ebuf, xbuf, acc, esem, gsem, ssem, isem):
            _body_common(x_hbm, src_hbm, dst_hbm, e_hbm, zeros_hbm, out_hbm,
                         srcb, dstb, ebuf, xbuf, acc, esem, gsem, ssem, isem)
    else:
        @deco
        def sc_agg(x_hbm, src_hbm, dst_hbm, e_hbm, zeros_hbm, out_hbm,
                   srcb, dstb, ebuf, acc, esem, gsem, ssem, isem):
            _body_common(x_hbm, src_hbm, dst_hbm, e_hbm, zeros_hbm, out_hbm,
                         srcb, dstb, ebuf, None, acc, esem, gsem, ssem, isem)

    return sc_agg


_sc_agg = functools.lru_cache(maxsize=None)(_make_sc_agg)


NG_NODE = N_NODES // BN


def _edge_feat_body(ea_ref, We1_ref, be1_ref, We2_ref, be2_ref, e1_ref, e2_ref):
    ea = ea_ref[...]
    e1_ref[...] = jnp.dot(ea, We1_ref[...],
                          preferred_element_type=jnp.float32) + be1_ref[...]
    e2_ref[...] = jnp.dot(ea, We2_ref[...],
                          preferred_element_type=jnp.float32) + be2_ref[...]


def _edge_feats(edge_attr, We1, be1, We2, be2):
    grid = N_EDGES // EB
    return pl.pallas_call(
        _edge_feat_body,
        grid=(grid,),
        in_specs=[
            pl.BlockSpec((EB, D_EDGE), lambda i: (i, 0)),
            pl.BlockSpec((D_EDGE, D_NODE), lambda i: (0, 0)),
            pl.BlockSpec((1, D_NODE), lambda i: (0, 0)),
            pl.BlockSpec((D_EDGE, 64), lambda i: (0, 0)),
            pl.BlockSpec((1, 64), lambda i: (0, 0)),
        ],
        out_specs=[
            pl.BlockSpec((EB, D_NODE), lambda i: (i, 0)),
            pl.BlockSpec((EB, 64), lambda i: (i, 0)),
        ],
        out_shape=[
            jax.ShapeDtypeStruct((N_EDGES, D_NODE), jnp.float32),
            jax.ShapeDtypeStruct((N_EDGES, 64), jnp.float32),
        ],
    )(edge_attr.astype(jnp.bfloat16), We1.astype(jnp.bfloat16), be1,
      We2.astype(jnp.bfloat16), be2)


def _bn_scale_shift(sa_ref, ssa_ref, g_ref, b_ref):
    mean = sa_ref[0:1, :] * (1.0 / N_NODES)
    var = ssa_ref[0:1, :] * (1.0 / N_NODES) - mean * mean
    inv = lax.rsqrt(var + BN_EPS)
    scale = inv * g_ref[...]
    shift = b_ref[...] - mean * scale
    return scale, shift


def _mlp_body(x_ref, a0_ref, a1_ref, W1_ref, b1_ref, W2_ref, b2_ref, hs_ref,
              sa_ref, ssa_ref):
    i = pl.program_id(0)

    @pl.when(i == 0)
    def _():
        sa_ref[...] = jnp.zeros_like(sa_ref)
        ssa_ref[...] = jnp.zeros_like(ssa_ref)

    u = x_ref[...] + a0_ref[...] + a1_ref[...]
    t = jnp.maximum(
        jnp.dot(u, W1_ref[...], preferred_element_type=jnp.float32)
        + b1_ref[...], 0.0)
    h = jnp.dot(t, W2_ref[...], preferred_element_type=jnp.float32) + b2_ref[...]
    hs_ref[pl.ds(i * BN, BN), :] = h
    sa_ref[...] += jnp.broadcast_to(jnp.sum(h, 0, keepdims=True), sa_ref.shape)
    ssa_ref[...] += jnp.broadcast_to(jnp.sum(h * h, 0, keepdims=True),
                                     ssa_ref.shape)


def _mlp_bn_body(x_ref, a0_ref, a1_ref, W1_ref, b1_ref, W2_ref, b2_ref,
                 g_ref, bt_ref, o_ref, hs_ref, sa_ref, ssa_ref):
    i = pl.program_id(0)

    @pl.when(i < NG_NODE)
    def _():
        _mlp_body(x_ref, a0_ref, a1_ref, W1_ref, b1_ref, W2_ref, b2_ref,
                  hs_ref, sa_ref, ssa_ref)

    @pl.when(i >= NG_NODE)
    def _():
        j = i - NG_NODE
        scale, shift = _bn_scale_shift(sa_ref, ssa_ref, g_ref, bt_ref)
        hb = hs_ref[pl.ds(j * BN, BN), :]
        o_ref[...] = jnp.maximum(hb * scale + shift, 0.0)


def _mlp_bn(x, a0, a1, W1, b1, W2, b2, g, bt):
    din = x.shape[1]
    dmid = W1.shape[1]
    dout = W2.shape[1]
    blk = lambda i: (jnp.where(i < NG_NODE, i, 0), 0)
    cst = lambda i: (0, 0)
    return pl.pallas_call(
        _mlp_bn_body,
        grid=(2 * NG_NODE,),
        in_specs=[
            pl.BlockSpec((BN, din), blk),
            pl.BlockSpec((BN, din), blk),
            pl.BlockSpec((BN, din), blk),
            pl.BlockSpec((din, dmid), cst),
            pl.BlockSpec((1, dmid), cst),
            pl.BlockSpec((dmid, dout), cst),
            pl.BlockSpec((1, dout), cst),
            pl.BlockSpec((1, dout), cst),
            pl.BlockSpec((1, dout), cst),
        ],
        out_specs=pl.BlockSpec((BN, dout),
                               lambda i: (jnp.where(i < NG_NODE, 0,
                                                    i - NG_NODE), 0)),
        out_shape=jax.ShapeDtypeStruct((N_NODES, dout), jnp.float32),
        scratch_shapes=[
            pltpu.VMEM((N_NODES, dout), jnp.float32),
            pltpu.VMEM((8, dout), jnp.float32),
            pltpu.VMEM((8, dout), jnp.float32),
        ],
    )(x, a0, a1, W1, b1, W2, b2, g, bt)


def _mlp_bn_pool_head_body(x_ref, a0_ref, a1_ref, W1_ref, b1_ref, W2_ref,
                           b2_ref, g_ref, bt_ref, batch_ref, gf_ref, Wa_ref,
                           Wb_ref, bf1_ref, Wf2_ref, bf2_ref, o_ref,
                           hs_ref, sa_ref, ssa_ref, ps_ref, cnt_ref):
    i = pl.program_id(0)

    @pl.when(i == 0)
    def _():
        ps_ref[...] = jnp.zeros_like(ps_ref)
        cnt_ref[...] = jnp.zeros_like(cnt_ref)

    @pl.when(i < NG_NODE)
    def _():
        _mlp_body(x_ref, a0_ref, a1_ref, W1_ref, b1_ref, W2_ref, b2_ref,
                  hs_ref, sa_ref, ssa_ref)

    @pl.when(i >= NG_NODE)
    def _():
        j = i - NG_NODE
        scale, shift = _bn_scale_shift(sa_ref, ssa_ref, g_ref, bt_ref)
        hb = hs_ref[pl.ds(j * BN, BN), :]
        t = jnp.maximum(hb * scale + shift, 0.0)
        bt_blk = batch_ref[0, 0, :]
        onehot = (lax.broadcasted_iota(jnp.int32, (N_GRAPHS, BN), 0)
                  == bt_blk[None, :]).astype(jnp.float32)
        ps_ref[...] += jnp.dot(onehot, t, preferred_element_type=jnp.float32)
        cnt_ref[...] += jnp.broadcast_to(jnp.sum(onehot, 1)[:, None],
                                         cnt_ref.shape)

    @pl.when(i == 2 * NG_NODE - 1)
    def _():
        cnt = jnp.maximum(cnt_ref[:, 0:1], 1.0)
        pooled = ps_ref[...] / cnt
        z = (jnp.dot(pooled, Wa_ref[...], preferred_element_type=jnp.float32)
             + jnp.dot(gf_ref[...], Wb_ref[...],
                       preferred_element_type=jnp.float32)
             + bf1_ref[...])
        z = jnp.maximum(z, 0.0)
        o_ref[...] = jnp.dot(z, Wf2_ref[...],
                             preferred_element_type=jnp.float32) + bf2_ref[...]


def _mlp_bn_pool_head(x, a0, a1, W1, b1, W2, b2, g, bt, batch3, gf, Wa, Wb,
                      bf1, Wf2, bf2):
    din = x.shape[1]
    dmid = W1.shape[1]
    dout = W2.shape[1]
    blk = lambda i: (jnp.where(i < NG_NODE, i, 0), 0)
    cst = lambda i: (0, 0)
    return pl.pallas_call(
        _mlp_bn_pool_head_body,
        grid=(2 * NG_NODE,),
        in_specs=[
            pl.BlockSpec((BN, din), blk),
            pl.BlockSpec((BN, din), blk),
            pl.BlockSpec((BN, din), blk),
            pl.BlockSpec((din, dmid), cst),
            pl.BlockSpec((1, dmid), cst),
            pl.BlockSpec((dmid, dout), cst),
            pl.BlockSpec((1, dout), cst),
            pl.BlockSpec((1, dout), cst),
            pl.BlockSpec((1, dout), cst),
            pl.BlockSpec((1, 1, BN),
                         lambda i: (jnp.where(i < NG_NODE, 0, i - NG_NODE),
                                    0, 0)),
            pl.BlockSpec((N_GRAPHS, D_GLOBAL), cst),
            pl.BlockSpec((dout, 128), cst),
            pl.BlockSpec((D_GLOBAL, 128), cst),
            pl.BlockSpec((1, 128), cst),
            pl.BlockSpec((128, 1), cst),
            pl.BlockSpec((1, 1), cst),
        ],
        out_specs=pl.BlockSpec((N_GRAPHS, 1), cst),
        out_shape=jax.ShapeDtypeStruct((N_GRAPHS, 1), jnp.float32),
        scratch_shapes=[
            pltpu.VMEM((N_NODES, dout), jnp.float32),
            pltpu.VMEM((8, dout), jnp.float32),
            pltpu.VMEM((8, dout), jnp.float32),
            pltpu.VMEM((N_GRAPHS, dout), jnp.float32),
            pltpu.VMEM((N_GRAPHS, 8), jnp.float32),
        ],
    )(x, a0, a1, W1, b1, W2, b2, g, bt, batch3, gf, Wa, Wb, bf1, Wf2, bf2)


def kernel(x, edge_index, edge_attr, batch, global_feat, We1, be1, W11, b11,
           W12, b12, g1, bt1, We2, be2, W21, b21, W22, b22, g2, bt2, Wf1,
           bf1, Wf2, bf2):
    src = edge_index[0].reshape(NW * NCH, C)
    dst = edge_index[1].reshape(NW * NCH, C)
    batch3 = batch.reshape(N_NODES // BN, 1, BN)
    zeros128 = jnp.zeros((N_NODES, 128), jnp.float32)
    zeros64 = jnp.zeros((N_NODES, 64), jnp.float32)

    e1, e2 = _edge_feats(edge_attr, We1, be1.reshape(1, -1),
                         We2, be2.reshape(1, -1))
    agg1 = _sc_agg(128, False)(x, src, dst, e1, zeros128)

    hn1 = _mlp_bn(x, agg1[0], agg1[1], W11, b11.reshape(1, -1),
                  W12, b12.reshape(1, -1), g1.reshape(1, -1),
                  bt1.reshape(1, -1))

    agg2 = _sc_agg(64, False)(hn1, src, dst, e2, zeros64)
    out = _mlp_bn_pool_head(hn1, agg2[0], agg2[1], W21, b21.reshape(1, -1),
                            W22, b22.reshape(1, -1), g2.reshape(1, -1),
                            bt2.reshape(1, -1), batch3, global_feat,
                            Wf1[:128], Wf1[128:], bf1.reshape(1, -1),
                            Wf2, bf2.reshape(1, -1))
    return out.reshape(N_GRAPHS)
